# Initial kernel scaffold; baseline (speedup 1.0000x reference)
import jax
import jax.numpy as jnp
from jax.experimental import pallas as pl


def kernel(query_embed, ctrl_points, bev_features, pc_range, Wq, bq, Wso, bso, Waw, baw, Wv, bv, Wmo, bmo, Wo, bo, spatial_shapes):
    raise NotImplementedError("write your pallas kernel here")


# speedup vs baseline: 4.6391x; 4.6391x over previous
import functools

import jax
import jax.numpy as jnp
from jax import lax
from jax.experimental import pallas as pl
from jax.experimental.pallas import tpu as pltpu
from jax.experimental.pallas import tpu_sc as plsc

HEADS = 8
POINTS = 4
HD = 32
BQ = 256


def _stage_a_body(qe_ref, ctrl_ref, wq_ref, bq_ref, wso_ref, bso_ref,
                  waw_ref, baw_ref, pc_ref, sp_ref,
                  q_out_ref, idx_out_ref, w_out_ref):
    f32 = jnp.float32
    bf16 = jnp.bfloat16
    q = jnp.dot(qe_ref[...].astype(bf16), wq_ref[...].astype(bf16),
                preferred_element_type=f32) + bq_ref[...]
    q_out_ref[...] = q

    qb = q.astype(bf16)
    so = jnp.dot(qb, wso_ref[...].astype(bf16), preferred_element_type=f32) + bso_ref[...]
    awl = jnp.dot(qb, waw_ref[...].astype(bf16), preferred_element_type=f32) + baw_ref[...]
    awl = awl - jnp.max(awl, axis=1, keepdims=True)
    e = jnp.exp(awl)
    col = lax.broadcasted_iota(jnp.int32, (HEADS * POINTS, HEADS * POINTS), 0)
    row = lax.broadcasted_iota(jnp.int32, (HEADS * POINTS, HEADS * POINTS), 1)
    gmask = (col // POINTS == row // POINTS).astype(f32)
    gsum = jnp.dot(e, gmask, preferred_element_type=f32, precision=jax.lax.Precision.HIGHEST)
    aw = e / gsum

    pc0, pc1, pc3, pc4 = pc_ref[0], pc_ref[1], pc_ref[3], pc_ref[4]
    cx = ctrl_ref[:, 0:1].astype(bf16).astype(f32)
    cy = ctrl_ref[:, 1:2].astype(bf16).astype(f32)
    rx = jnp.clip((cx - pc0) / (pc3 - pc0), 0.01, 0.99)
    ry = jnp.clip((cy - pc1) / (pc4 - pc1), 0.01, 0.99)

    wn = sp_ref[0, 1].astype(f32)
    hn = sp_ref[0, 0].astype(f32)
    hs = sp_ref[0, 0]
    ws = sp_ref[0, 1]
    slx = rx + so[:, 0:32] / wn
    sly = ry + so[:, 32:64] / hn
    gx = (2.0 * slx - 1.0 + 1.0) * wn / 2.0 - 0.5
    gy = (2.0 * sly - 1.0 + 1.0) * hn / 2.0 - 0.5
    x0 = jnp.floor(gx)
    y0 = jnp.floor(gy)
    wx1 = gx - x0
    wx0 = 1.0 - wx1
    wy1 = gy - y0
    wy0 = 1.0 - wy1

    hcol = lax.broadcasted_iota(jnp.int32, (BQ, HEADS * POINTS), 1) // POINTS

    def corner(xo, yo, wx, wy):
        ix = x0 + xo
        iy = y0 + yo
        wf = jnp.float32(1.0) * ws.astype(f32)
        hf = jnp.float32(1.0) * hs.astype(f32)
        valid = (ix >= 0.0) & (ix <= wf - 1.0) & (iy >= 0.0) & (iy <= hf - 1.0)
        ixc = jnp.clip(ix, 0.0, wf - 1.0).astype(jnp.int32)
        iyc = jnp.clip(iy, 0.0, hf - 1.0).astype(jnp.int32)
        ridx = (iyc * ws + ixc) * 2 + hcol // 4
        wgt = jnp.where(valid, wx * wy * aw, 0.0)
        return ridx, wgt

    i00, w00 = corner(0.0, 0.0, wx0, wy0)
    i10, w10 = corner(1.0, 0.0, wx1, wy0)
    i01, w01 = corner(0.0, 1.0, wx0, wy1)
    i11, w11 = corner(1.0, 1.0, wx1, wy1)
    idx_out_ref[...] = jnp.concatenate([i00, i10, i01, i11], axis=1)
    w_out_ref[...] = jnp.concatenate([w00, w10, w01, w11], axis=1)


def _stage_a(qe, ctrl8, wq, bq, wso_r, bso_r, waw, baw, pc, sp):
    nq, d = qe.shape
    grid = (nq // BQ,)
    return pl.pallas_call(
        _stage_a_body,
        grid=grid,
        in_specs=[
            pl.BlockSpec((BQ, d), lambda i: (i, 0)),
            pl.BlockSpec((BQ, 8), lambda i: (i, 0)),
            pl.BlockSpec((d, d), lambda i: (0, 0)),
            pl.BlockSpec((1, d), lambda i: (0, 0)),
            pl.BlockSpec((d, 2 * HEADS * POINTS), lambda i: (0, 0)),
            pl.BlockSpec((1, 2 * HEADS * POINTS), lambda i: (0, 0)),
            pl.BlockSpec((d, HEADS * POINTS), lambda i: (0, 0)),
            pl.BlockSpec((1, HEADS * POINTS), lambda i: (0, 0)),
            pl.BlockSpec(memory_space=pltpu.SMEM),
            pl.BlockSpec(memory_space=pltpu.SMEM),
        ],
        out_specs=[
            pl.BlockSpec((BQ, d), lambda i: (i, 0)),
            pl.BlockSpec((BQ, 4 * HEADS * POINTS), lambda i: (i, 0)),
            pl.BlockSpec((BQ, 4 * HEADS * POINTS), lambda i: (i, 0)),
        ],
        out_shape=[
            jax.ShapeDtypeStruct((nq, d), jnp.float32),
            jax.ShapeDtypeStruct((nq, 4 * HEADS * POINTS), jnp.int32),
            jax.ShapeDtypeStruct((nq, 4 * HEADS * POINTS), jnp.float32),
        ],
    )(qe, ctrl8, wq, bq, wso_r, bso_r, waw, baw, pc, sp)


def _value_body(bev_ref, wv_ref, bv_ref, out_ref):
    out_ref[...] = lax.dot_general(
        bev_ref[...].astype(jnp.bfloat16), wv_ref[...].astype(jnp.bfloat16),
        (((0,), (0,)), ((), ())),
        preferred_element_type=jnp.float32) + bv_ref[...]


def _value_project(bev_cm, wv, bv):
    c, npix = bev_cm.shape
    pb = 2048
    grid = (pl.cdiv(npix, pb),)
    return pl.pallas_call(
        _value_body,
        grid=grid,
        in_specs=[
            pl.BlockSpec((c, pb), lambda i: (0, i)),
            pl.BlockSpec((c, c), lambda i: (0, 0)),
            pl.BlockSpec((1, c), lambda i: (0, 0)),
        ],
        out_specs=pl.BlockSpec((pb, c), lambda i: (i, 0)),
        out_shape=jax.ShapeDtypeStruct((npix, c), jnp.float32),
    )(bev_cm, wv, bv)


def _stage_c_body(ms_ref, q_ref, wmo_ref, bmo_ref, wo_ref, bo_ref, out_ref):
    f32 = jnp.float32
    bf16 = jnp.bfloat16
    h1 = jnp.dot(ms_ref[...].astype(bf16), wmo_ref[...].astype(bf16),
                 preferred_element_type=f32)
    h1 = h1 + bmo_ref[...] + q_ref[...]
    out_ref[...] = jnp.dot(h1.astype(bf16), wo_ref[...].astype(bf16),
                           preferred_element_type=f32) + bo_ref[...]


def _stage_c(msda, q, wmo, bmo, wo, bo):
    nq, d = msda.shape
    grid = (nq // BQ,)
    return pl.pallas_call(
        _stage_c_body,
        grid=grid,
        in_specs=[
            pl.BlockSpec((BQ, d), lambda i: (i, 0)),
            pl.BlockSpec((BQ, d), lambda i: (i, 0)),
            pl.BlockSpec((d, d), lambda i: (0, 0)),
            pl.BlockSpec((1, d), lambda i: (0, 0)),
            pl.BlockSpec((d, d), lambda i: (0, 0)),
            pl.BlockSpec((1, d), lambda i: (0, 0)),
        ],
        out_specs=pl.BlockSpec((BQ, d), lambda i: (i, 0)),
        out_shape=jax.ShapeDtypeStruct((nq, d), jnp.float32),
    )(msda, q, wmo, bmo, wo, bo)


_NW = 32
_RPQ = 4 * HEADS * POINTS


def _sc_gather_combine(table, idx_flat, w_flat, nq):
    qpw = nq // _NW
    mesh = plsc.VectorSubcoreMesh(
        core_axis_name="c", subcore_axis_name="s", num_cores=2, num_subcores=16)

    @functools.partial(
        pl.kernel,
        out_type=jax.ShapeDtypeStruct((nq * HEADS, HD), jnp.float32),
        mesh=mesh,
        scratch_types=[
            pltpu.VMEM((qpw * _RPQ,), jnp.int32),
            pltpu.VMEM((qpw * _RPQ,), jnp.float32),
            pltpu.VMEM((_RPQ, 128), jnp.float32),
            pltpu.VMEM((qpw * HEADS, HD), jnp.float32),
            pltpu.SemaphoreType.DMA,
        ],
    )
    def k(table_hbm, idx_hbm, w_hbm, out_hbm, idx_v, w_v, rows_v, out_v, sem):
        wid = lax.axis_index("s") * 2 + lax.axis_index("c")
        ebase = wid * (qpw * _RPQ)
        pltpu.sync_copy(idx_hbm.at[pl.ds(ebase, qpw * _RPQ)], idx_v)
        pltpu.sync_copy(w_hbm.at[pl.ds(ebase, qpw * _RPQ)], w_v)

        def body(qi, carry):
            pltpu.async_copy(
                table_hbm.at[idx_v.at[pl.ds(qi * _RPQ, _RPQ)]], rows_v, sem
            ).wait()
            wvs = [w_v[pl.ds(qi * _RPQ + k * 16, 16)] for k in range(_RPQ // 16)]
            for h in range(HEADS):
                acc0 = jnp.zeros((16,), jnp.float32)
                acc1 = jnp.zeros((16,), jnp.float32)
                ho = (h % 4) * HD
                for c in range(4):
                    for p in range(POINTS):
                        r = c * HEADS * POINTS + h * POINTS + p
                        wgt = wvs[r // 16][r % 16]
                        acc0 = acc0 + wgt * rows_v[r, pl.ds(ho, 16)]
                        acc1 = acc1 + wgt * rows_v[r, pl.ds(ho + 16, 16)]
                out_v[qi * HEADS + h, pl.ds(0, 16)] = acc0
                out_v[qi * HEADS + h, pl.ds(16, 16)] = acc1
            return carry

        lax.fori_loop(0, qpw, body, 0)
        pltpu.sync_copy(out_v, out_hbm.at[pl.ds(wid * qpw * HEADS, qpw * HEADS)])

    return k(table, idx_flat, w_flat)


def kernel(query_embed, ctrl_points, bev_features, pc_range, Wq, bq, Wso, bso,
           Waw, baw, Wv, bv, Wmo, bmo, Wo, bo, spatial_shapes):
    b, nq, d = query_embed.shape
    _, c, h, w = bev_features.shape
    qe = query_embed.reshape(b * nq, d)
    ctrl8 = ctrl_points.reshape(b * nq, 8)
    wso_r = Wso.reshape(d, HEADS, POINTS, 2).transpose(0, 3, 1, 2).reshape(d, 2 * HEADS * POINTS)
    bso_r = bso.reshape(HEADS, POINTS, 2).transpose(2, 0, 1).reshape(1, 2 * HEADS * POINTS)

    qp, idx128, w128 = _stage_a(
        qe, ctrl8, Wq, bq.reshape(1, d), wso_r, bso_r,
        Waw, baw.reshape(1, HEADS * POINTS), pc_range, spatial_shapes)

    value = _value_project(bev_features.reshape(c, h * w), Wv, bv.reshape(1, d))
    table = value.reshape(h * w * 2, 128)

    msda = _sc_gather_combine(table, idx128.reshape(-1), w128.reshape(-1), b * nq)

    out = _stage_c(msda.reshape(b * nq, d), qp, Wmo, bmo.reshape(1, d),
                   Wo, bo.reshape(1, d))
    return out.reshape(b, nq, d)


# speedup vs baseline: 5.4339x; 1.1713x over previous
import functools

import jax
import jax.numpy as jnp
from jax import lax
from jax.experimental import pallas as pl
from jax.experimental.pallas import tpu as pltpu
from jax.experimental.pallas import tpu_sc as plsc

HEADS = 8
POINTS = 4
HD = 32
BQ = 256


def _stage_a_body(qe_ref, ctrl_ref, wq_ref, bq_ref, wso_ref, bso_ref,
                  waw_ref, baw_ref, pc_ref, sp_ref,
                  q_out_ref, idx_out_ref, w_out_ref):
    f32 = jnp.float32
    bf16 = jnp.bfloat16
    q = jnp.dot(qe_ref[...].astype(bf16), wq_ref[...].astype(bf16),
                preferred_element_type=f32) + bq_ref[...]
    q_out_ref[...] = q

    qb = q.astype(bf16)
    so = jnp.dot(qb, wso_ref[...].astype(bf16), preferred_element_type=f32) + bso_ref[...]
    awl = jnp.dot(qb, waw_ref[...].astype(bf16), preferred_element_type=f32) + baw_ref[...]
    awl = awl - jnp.max(awl, axis=1, keepdims=True)
    e = jnp.exp(awl)
    col = lax.broadcasted_iota(jnp.int32, (HEADS * POINTS, HEADS * POINTS), 0)
    row = lax.broadcasted_iota(jnp.int32, (HEADS * POINTS, HEADS * POINTS), 1)
    gmask = (col // POINTS == row // POINTS).astype(f32)
    gsum = jnp.dot(e, gmask, preferred_element_type=f32, precision=jax.lax.Precision.HIGHEST)
    aw = e / gsum

    pc0, pc1, pc3, pc4 = pc_ref[0], pc_ref[1], pc_ref[3], pc_ref[4]
    cx = ctrl_ref[:, 0:1].astype(bf16).astype(f32)
    cy = ctrl_ref[:, 1:2].astype(bf16).astype(f32)
    rx = jnp.clip((cx - pc0) / (pc3 - pc0), 0.01, 0.99)
    ry = jnp.clip((cy - pc1) / (pc4 - pc1), 0.01, 0.99)

    wn = sp_ref[0, 1].astype(f32)
    hn = sp_ref[0, 0].astype(f32)
    hs = sp_ref[0, 0]
    ws = sp_ref[0, 1]
    slx = rx + so[:, 0:32] / wn
    sly = ry + so[:, 32:64] / hn
    gx = (2.0 * slx - 1.0 + 1.0) * wn / 2.0 - 0.5
    gy = (2.0 * sly - 1.0 + 1.0) * hn / 2.0 - 0.5
    x0 = jnp.floor(gx)
    y0 = jnp.floor(gy)
    wx1 = gx - x0
    wx0 = 1.0 - wx1
    wy1 = gy - y0
    wy0 = 1.0 - wy1

    hcol = lax.broadcasted_iota(jnp.int32, (BQ, HEADS * POINTS), 1) // POINTS

    def corner(xo, yo, wx, wy):
        ix = x0 + xo
        iy = y0 + yo
        wf = jnp.float32(1.0) * ws.astype(f32)
        hf = jnp.float32(1.0) * hs.astype(f32)
        valid = (ix >= 0.0) & (ix <= wf - 1.0) & (iy >= 0.0) & (iy <= hf - 1.0)
        ixc = jnp.clip(ix, 0.0, wf - 1.0).astype(jnp.int32)
        iyc = jnp.clip(iy, 0.0, hf - 1.0).astype(jnp.int32)
        ridx = (iyc * ws + ixc) * 2 + hcol // 4
        wgt = jnp.where(valid, wx * wy * aw, 0.0)
        return ridx, wgt

    i00, w00 = corner(0.0, 0.0, wx0, wy0)
    i10, w10 = corner(1.0, 0.0, wx1, wy0)
    i01, w01 = corner(0.0, 1.0, wx0, wy1)
    i11, w11 = corner(1.0, 1.0, wx1, wy1)
    idx_out_ref[...] = jnp.concatenate([i00, i10, i01, i11], axis=1)
    w_out_ref[...] = jnp.concatenate([w00, w10, w01, w11], axis=1)


def _stage_a(qe, ctrl8, wq, bq, wso_r, bso_r, waw, baw, pc, sp):
    nq, d = qe.shape
    grid = (nq // BQ,)
    return pl.pallas_call(
        _stage_a_body,
        grid=grid,
        in_specs=[
            pl.BlockSpec((BQ, d), lambda i: (i, 0)),
            pl.BlockSpec((BQ, 8), lambda i: (i, 0)),
            pl.BlockSpec((d, d), lambda i: (0, 0)),
            pl.BlockSpec((1, d), lambda i: (0, 0)),
            pl.BlockSpec((d, 2 * HEADS * POINTS), lambda i: (0, 0)),
            pl.BlockSpec((1, 2 * HEADS * POINTS), lambda i: (0, 0)),
            pl.BlockSpec((d, HEADS * POINTS), lambda i: (0, 0)),
            pl.BlockSpec((1, HEADS * POINTS), lambda i: (0, 0)),
            pl.BlockSpec(memory_space=pltpu.SMEM),
            pl.BlockSpec(memory_space=pltpu.SMEM),
        ],
        out_specs=[
            pl.BlockSpec((BQ, d), lambda i: (i, 0)),
            pl.BlockSpec((BQ, 4 * HEADS * POINTS), lambda i: (i, 0)),
            pl.BlockSpec((BQ, 4 * HEADS * POINTS), lambda i: (i, 0)),
        ],
        out_shape=[
            jax.ShapeDtypeStruct((nq, d), jnp.float32),
            jax.ShapeDtypeStruct((nq, 4 * HEADS * POINTS), jnp.int32),
            jax.ShapeDtypeStruct((nq, 4 * HEADS * POINTS), jnp.float32),
        ],
    )(qe, ctrl8, wq, bq, wso_r, bso_r, waw, baw, pc, sp)


def _value_body(bev_ref, wv_ref, bv_ref, out_ref):
    out_ref[...] = lax.dot_general(
        bev_ref[...].astype(jnp.bfloat16), wv_ref[...].astype(jnp.bfloat16),
        (((0,), (0,)), ((), ())),
        preferred_element_type=jnp.float32) + bv_ref[...]


def _value_project(bev_cm, wv, bv):
    c, npix = bev_cm.shape
    pb = 2048
    grid = (pl.cdiv(npix, pb),)
    return pl.pallas_call(
        _value_body,
        grid=grid,
        in_specs=[
            pl.BlockSpec((c, pb), lambda i: (0, i)),
            pl.BlockSpec((c, c), lambda i: (0, 0)),
            pl.BlockSpec((1, c), lambda i: (0, 0)),
        ],
        out_specs=pl.BlockSpec((pb, c), lambda i: (i, 0)),
        out_shape=jax.ShapeDtypeStruct((npix, c), jnp.float32),
    )(bev_cm, wv, bv)


def _stage_c_body(ms_ref, q_ref, wmo_ref, bmo_ref, wo_ref, bo_ref, out_ref):
    f32 = jnp.float32
    bf16 = jnp.bfloat16
    h1 = jnp.dot(ms_ref[...].astype(bf16), wmo_ref[...].astype(bf16),
                 preferred_element_type=f32)
    h1 = h1 + bmo_ref[...] + q_ref[...]
    out_ref[...] = jnp.dot(h1.astype(bf16), wo_ref[...].astype(bf16),
                           preferred_element_type=f32) + bo_ref[...]


def _stage_c(msda, q, wmo, bmo, wo, bo):
    nq, d = msda.shape
    grid = (nq // BQ,)
    return pl.pallas_call(
        _stage_c_body,
        grid=grid,
        in_specs=[
            pl.BlockSpec((BQ, d), lambda i: (i, 0)),
            pl.BlockSpec((BQ, d), lambda i: (i, 0)),
            pl.BlockSpec((d, d), lambda i: (0, 0)),
            pl.BlockSpec((1, d), lambda i: (0, 0)),
            pl.BlockSpec((d, d), lambda i: (0, 0)),
            pl.BlockSpec((1, d), lambda i: (0, 0)),
        ],
        out_specs=pl.BlockSpec((BQ, d), lambda i: (i, 0)),
        out_shape=jax.ShapeDtypeStruct((nq, d), jnp.float32),
    )(msda, q, wmo, bmo, wo, bo)


_NW = 32
_RPQ = 4 * HEADS * POINTS


def _sc_gather_combine(table, idx_flat, w_flat, nq):
    qpw = nq // _NW
    mesh = plsc.VectorSubcoreMesh(
        core_axis_name="c", subcore_axis_name="s", num_cores=2, num_subcores=16)

    chq = 2
    chr_ = chq * _RPQ
    nch = qpw // chq

    @functools.partial(
        pl.kernel,
        out_type=jax.ShapeDtypeStruct((nq, HEADS * HD), jnp.float32),
        mesh=mesh,
        scratch_types=[
            pltpu.VMEM((qpw * _RPQ,), jnp.int32),
            pltpu.VMEM((qpw * _RPQ,), jnp.float32),
            pltpu.VMEM((chr_, 128), jnp.float32),
            pltpu.VMEM((chr_, 128), jnp.float32),
            pltpu.VMEM((qpw, HEADS * HD), jnp.float32),
            pltpu.SemaphoreType.DMA,
            pltpu.SemaphoreType.DMA,
        ],
    )
    def k(table_hbm, idx_hbm, w_hbm, out_hbm, idx_v, w_v, rows0_v, rows1_v,
          out_v, sem0, sem1):
        wid = lax.axis_index("s") * 2 + lax.axis_index("c")
        ebase = wid * (qpw * _RPQ)
        pltpu.sync_copy(idx_hbm.at[pl.ds(ebase, qpw * _RPQ)], idx_v)
        pltpu.sync_copy(w_hbm.at[pl.ds(ebase, qpw * _RPQ)], w_v)

        def start(ch, buf, sem):
            pltpu.async_copy(
                table_hbm.at[idx_v.at[pl.ds(ch * chr_, chr_)]], buf, sem)

        def waitbuf(buf, sem):
            pltpu.make_async_copy(table_hbm.at[pl.ds(0, chr_)], buf, sem).wait()

        def combine(ch, buf):
            wvs = [w_v[pl.ds(ch * chr_ + k * 16, 16)] for k in range(chr_ // 16)]
            for qq in range(chq):
                for h in range(HEADS):
                    acc0 = jnp.zeros((16,), jnp.float32)
                    acc1 = jnp.zeros((16,), jnp.float32)
                    ho = (h % 4) * HD
                    for c in range(4):
                        for p in range(POINTS):
                            r = qq * _RPQ + c * HEADS * POINTS + h * POINTS + p
                            wgt = wvs[r // 16][r % 16]
                            acc0 = acc0 + wgt * buf[r, pl.ds(ho, 16)]
                            acc1 = acc1 + wgt * buf[r, pl.ds(ho + 16, 16)]
                    o = ch * chq + qq
                    out_v[o, pl.ds(h * HD, 16)] = acc0
                    out_v[o, pl.ds(h * HD + 16, 16)] = acc1

        start(0, rows0_v, sem0)

        def body(g, carry):
            start(2 * g + 1, rows1_v, sem1)
            waitbuf(rows0_v, sem0)
            combine(2 * g, rows0_v)

            @pl.when(g < nch // 2 - 1)
            def _():
                start(2 * g + 2, rows0_v, sem0)

            waitbuf(rows1_v, sem1)
            combine(2 * g + 1, rows1_v)
            return carry

        lax.fori_loop(0, nch // 2, body, 0)
        pltpu.sync_copy(out_v, out_hbm.at[pl.ds(wid * qpw, qpw)])

    return k(table, idx_flat, w_flat)


def kernel(query_embed, ctrl_points, bev_features, pc_range, Wq, bq, Wso, bso,
           Waw, baw, Wv, bv, Wmo, bmo, Wo, bo, spatial_shapes):
    b, nq, d = query_embed.shape
    _, c, h, w = bev_features.shape
    qe = query_embed.reshape(b * nq, d)
    ctrl8 = ctrl_points.reshape(b * nq, 8)
    wso_r = Wso.reshape(d, HEADS, POINTS, 2).transpose(0, 3, 1, 2).reshape(d, 2 * HEADS * POINTS)
    bso_r = bso.reshape(HEADS, POINTS, 2).transpose(2, 0, 1).reshape(1, 2 * HEADS * POINTS)

    qp, idx128, w128 = _stage_a(
        qe, ctrl8, Wq, bq.reshape(1, d), wso_r, bso_r,
        Waw, baw.reshape(1, HEADS * POINTS), pc_range, spatial_shapes)

    value = _value_project(bev_features.reshape(c, h * w), Wv, bv.reshape(1, d))
    table = value.reshape(h * w * 2, 128)

    msda = _sc_gather_combine(table, idx128.reshape(-1), w128.reshape(-1), b * nq)

    out = _stage_c(msda, qp, Wmo, bmo.reshape(1, d),
                   Wo, bo.reshape(1, d))
    return out.reshape(b, nq, d)
